# initial kernel scaffold (unmeasured)
import jax
import jax.numpy as jnp
from jax import lax
from jax.experimental import pallas as pl
from jax.experimental.pallas import tpu as pltpu

N_DEV = 4
COMM_DTYPE = jnp.bfloat16


def kernel(x, w_mat, scale_x, scale_w):
    m_total, k_shard = x.shape
    _, n_out = w_mat.shape
    m_chunk = m_total // N_DEV

    def body(x_ref, w_ref, sx_ref, sw_ref, out_ref,
             send_buf, recv_buf, send_sems, recv_sems):
        my = lax.axis_index("i")
        left = (my + N_DEV - 1) % N_DEV
        right = (my + 1) % N_DEV

        barrier_sem = pltpu.get_barrier_semaphore()
        for nbr in (left, right):
            pl.semaphore_signal(
                barrier_sem, inc=1,
                device_id=(nbr,), device_id_type=pl.DeviceIdType.MESH,
            )
        pl.semaphore_wait(barrier_sem, 2)

        def pchunk(c):
            rows = x_ref[pl.ds(c * m_chunk, m_chunk), :]
            return jnp.dot(rows, w_ref[...], preferred_element_type=jnp.float32)

        send_buf[...] = pchunk((my + N_DEV - 1) % N_DEV).astype(COMM_DTYPE)

        for h in range(N_DEV - 1):
            rdma = pltpu.make_async_remote_copy(
                src_ref=send_buf,
                dst_ref=recv_buf.at[h],
                send_sem=send_sems.at[h],
                recv_sem=recv_sems.at[h],
                device_id=(right,),
                device_id_type=pl.DeviceIdType.MESH,
            )
            rdma.start()
            rdma.wait()
            if h < N_DEV - 2:
                c = (my + N_DEV - 2 - h) % N_DEV
                acc = pchunk(c) + recv_buf[h].astype(jnp.float32)
                send_buf[...] = acc.astype(COMM_DTYPE)
            else:
                acc = pchunk(my) + recv_buf[h].astype(jnp.float32)
                y = acc * (sx_ref[0] * sw_ref[0])
                z = jnp.clip(y, -60.0, 60.0)
                out_ref[...] = y / (1.0 + jnp.exp(-z))

    return pl.pallas_call(
        body,
        out_shape=jax.ShapeDtypeStruct((m_chunk, n_out), jnp.float32),
        in_specs=[
            pl.BlockSpec(memory_space=pltpu.VMEM),
            pl.BlockSpec(memory_space=pltpu.VMEM),
            pl.BlockSpec(memory_space=pltpu.SMEM),
            pl.BlockSpec(memory_space=pltpu.SMEM),
        ],
        out_specs=pl.BlockSpec(memory_space=pltpu.VMEM),
        scratch_shapes=[
            pltpu.VMEM((m_chunk, n_out), COMM_DTYPE),
            pltpu.VMEM((N_DEV - 1, m_chunk, n_out), COMM_DTYPE),
            pltpu.SemaphoreType.DMA((N_DEV - 1,)),
            pltpu.SemaphoreType.DMA((N_DEV - 1,)),
        ],
        compiler_params=pltpu.CompilerParams(collective_id=0),
    )(x, w_mat, scale_x, scale_w)


# baseline (device time: 182193 ns/iter reference)
import jax
import jax.numpy as jnp
from jax import lax
from jax.experimental import pallas as pl
from jax.experimental.pallas import tpu as pltpu

N_DEV = 4
COMM_DTYPE = jnp.bfloat16


def kernel(x, w_mat, scale_x, scale_w):
    m_total, k_shard = x.shape
    _, n_out = w_mat.shape
    m_chunk = m_total // N_DEV

    def body(x_ref, w_ref, sx_ref, sw_ref, out_ref,
             send_buf, recv_buf, send_sems, recv_sems):
        my = lax.axis_index("i")
        left = (my + N_DEV - 1) % N_DEV
        right = (my + 1) % N_DEV

        barrier_sem = pltpu.get_barrier_semaphore()
        for nbr in (left, right):
            pl.semaphore_signal(
                barrier_sem, inc=1,
                device_id=(nbr,), device_id_type=pl.DeviceIdType.MESH,
            )
        pl.semaphore_wait(barrier_sem, 2)

        def pchunk(c):
            rows = x_ref[pl.ds(c * m_chunk, m_chunk), :]
            return jnp.dot(rows, w_ref[...], preferred_element_type=jnp.float32)

        send_buf[...] = pchunk((my + N_DEV - 1) % N_DEV).astype(COMM_DTYPE)

        for h in range(N_DEV - 1):
            rdma = pltpu.make_async_remote_copy(
                src_ref=send_buf,
                dst_ref=recv_buf.at[h],
                send_sem=send_sems.at[h],
                recv_sem=recv_sems.at[h],
                device_id=(right,),
                device_id_type=pl.DeviceIdType.MESH,
            )
            rdma.start()
            rdma.wait()
            if h < N_DEV - 2:
                c = (my + N_DEV - 2 - h) % N_DEV
                acc = pchunk(c) + recv_buf[h].astype(jnp.float32)
                send_buf[...] = acc.astype(COMM_DTYPE)
            else:
                acc = pchunk(my) + recv_buf[h].astype(jnp.float32)
                y = acc * (sx_ref[0] * sw_ref[0])
                z = jnp.clip(y, -60.0, 60.0)
                out_ref[...] = y / (1.0 + jnp.exp(-z))

    return pl.pallas_call(
        body,
        out_shape=jax.ShapeDtypeStruct((m_chunk, n_out), jnp.float32),
        in_specs=[
            pl.BlockSpec(memory_space=pltpu.VMEM),
            pl.BlockSpec(memory_space=pltpu.VMEM),
            pl.BlockSpec(memory_space=pltpu.SMEM),
            pl.BlockSpec(memory_space=pltpu.SMEM),
        ],
        out_specs=pl.BlockSpec(memory_space=pltpu.VMEM),
        scratch_shapes=[
            pltpu.VMEM((m_chunk, n_out), COMM_DTYPE),
            pltpu.VMEM((N_DEV - 1, m_chunk, n_out), COMM_DTYPE),
            pltpu.SemaphoreType.DMA((N_DEV - 1,)),
            pltpu.SemaphoreType.DMA((N_DEV - 1,)),
        ],
        compiler_params=pltpu.CompilerParams(
            collective_id=0, vmem_limit_bytes=100 * 1024 * 1024
        ),
    )(x, w_mat, scale_x, scale_w)


# device time: 115360 ns/iter; 1.5793x vs baseline; 1.5793x over previous
import jax
import jax.numpy as jnp
from jax import lax
from jax.experimental import pallas as pl
from jax.experimental.pallas import tpu as pltpu

N_DEV = 4
COMM_DTYPE = jnp.bfloat16


def kernel(x, w_mat, scale_x, scale_w):
    m_total, k_shard = x.shape
    _, n_out = w_mat.shape
    m_chunk = m_total // N_DEV
    n_half = n_out // 2

    def body(x_ref, w_ref, sx_ref, sw_ref, out_ref,
             send_r, send_l, recv_r, recv_l,
             ss_r, rs_r, ss_l, rs_l):
        my = lax.axis_index("i")
        left = (my + N_DEV - 1) % N_DEV
        right = (my + 1) % N_DEV

        barrier_sem = pltpu.get_barrier_semaphore()
        for nbr in (left, right):
            pl.semaphore_signal(
                barrier_sem, inc=1,
                device_id=(nbr,), device_id_type=pl.DeviceIdType.MESH,
            )
        pl.semaphore_wait(barrier_sem, 2)

        def part_a(c):
            rows = x_ref[pl.ds(c * m_chunk, m_chunk), :]
            return jnp.dot(rows, w_ref[:, :n_half],
                           preferred_element_type=jnp.float32)

        def part_b(c):
            rows = x_ref[pl.ds(c * m_chunk, m_chunk), :]
            return jnp.dot(rows, w_ref[:, n_half:],
                           preferred_element_type=jnp.float32)

        def start_hop(h):
            r = pltpu.make_async_remote_copy(
                src_ref=send_r, dst_ref=recv_r.at[h],
                send_sem=ss_r.at[h], recv_sem=rs_r.at[h],
                device_id=(right,), device_id_type=pl.DeviceIdType.MESH,
            )
            l = pltpu.make_async_remote_copy(
                src_ref=send_l, dst_ref=recv_l.at[h],
                send_sem=ss_l.at[h], recv_sem=rs_l.at[h],
                device_id=(left,), device_id_type=pl.DeviceIdType.MESH,
            )
            r.start()
            l.start()
            return r, l

        send_r[...] = part_a((my + N_DEV - 1) % N_DEV).astype(COMM_DTYPE)
        send_l[...] = part_b((my + 1) % N_DEV).astype(COMM_DTYPE)
        cur_r, cur_l = start_hop(0)

        for h in range(N_DEV - 2):
            a = part_a((my + N_DEV - 2 - h) % N_DEV)
            b = part_b((my + 2 + h) % N_DEV)
            cur_r.wait_recv()
            cur_r.wait_send()
            send_r[...] = (a + recv_r[h].astype(jnp.float32)).astype(COMM_DTYPE)
            cur_l.wait_recv()
            cur_l.wait_send()
            send_l[...] = (b + recv_l[h].astype(jnp.float32)).astype(COMM_DTYPE)
            cur_r, cur_l = start_hop(h + 1)

        h = N_DEV - 2
        a = part_a(my)
        b = part_b(my)
        cur_r.wait_recv()
        cur_r.wait_send()
        cur_l.wait_recv()
        cur_l.wait_send()
        scale = sx_ref[0] * sw_ref[0]
        ya = (a + recv_r[h].astype(jnp.float32)) * scale
        yb = (b + recv_l[h].astype(jnp.float32)) * scale
        za = jnp.clip(ya, -60.0, 60.0)
        zb = jnp.clip(yb, -60.0, 60.0)
        out_ref[:, :n_half] = ya / (1.0 + jnp.exp(-za))
        out_ref[:, n_half:] = yb / (1.0 + jnp.exp(-zb))

    return pl.pallas_call(
        body,
        out_shape=jax.ShapeDtypeStruct((m_chunk, n_out), jnp.float32),
        in_specs=[
            pl.BlockSpec(memory_space=pltpu.VMEM),
            pl.BlockSpec(memory_space=pltpu.VMEM),
            pl.BlockSpec(memory_space=pltpu.SMEM),
            pl.BlockSpec(memory_space=pltpu.SMEM),
        ],
        out_specs=pl.BlockSpec(memory_space=pltpu.VMEM),
        scratch_shapes=[
            pltpu.VMEM((m_chunk, n_half), COMM_DTYPE),
            pltpu.VMEM((m_chunk, n_half), COMM_DTYPE),
            pltpu.VMEM((N_DEV - 1, m_chunk, n_half), COMM_DTYPE),
            pltpu.VMEM((N_DEV - 1, m_chunk, n_half), COMM_DTYPE),
            pltpu.SemaphoreType.DMA((N_DEV - 1,)),
            pltpu.SemaphoreType.DMA((N_DEV - 1,)),
            pltpu.SemaphoreType.DMA((N_DEV - 1,)),
            pltpu.SemaphoreType.DMA((N_DEV - 1,)),
        ],
        compiler_params=pltpu.CompilerParams(
            collective_id=0, vmem_limit_bytes=100 * 1024 * 1024
        ),
    )(x, w_mat, scale_x, scale_w)


# device time: 107224 ns/iter; 1.6992x vs baseline; 1.0759x over previous
import jax
import jax.numpy as jnp
from jax import lax
from jax.experimental import pallas as pl
from jax.experimental.pallas import tpu as pltpu

N_DEV = 4
COMM_DTYPE = jnp.bfloat16


def kernel(x, w_mat, scale_x, scale_w):
    m_total, k_shard = x.shape
    _, n_out = w_mat.shape
    m_chunk = m_total // N_DEV
    n_half = n_out // 2

    def body(x_ref, w_ref, sx_ref, sw_ref, out_ref,
             x8_ref, w8_ref,
             send_r, send_l, recv_r, recv_l,
             ss_r, rs_r, ss_l, rs_l):
        my = lax.axis_index("i")
        left = (my + N_DEV - 1) % N_DEV
        right = (my + 1) % N_DEV

        x8_ref[...] = x_ref[...].astype(jnp.float8_e4m3fn)
        w8_ref[...] = w_ref[...].astype(jnp.float8_e4m3fn)

        def part_a(c):
            rows = x8_ref[pl.ds(c * m_chunk, m_chunk), :]
            return jnp.dot(rows, w8_ref[:, :n_half],
                           preferred_element_type=jnp.float32)

        def part_b(c):
            rows = x8_ref[pl.ds(c * m_chunk, m_chunk), :]
            return jnp.dot(rows, w8_ref[:, n_half:],
                           preferred_element_type=jnp.float32)

        def start_hop(h):
            r = pltpu.make_async_remote_copy(
                src_ref=send_r, dst_ref=recv_r.at[h],
                send_sem=ss_r.at[h], recv_sem=rs_r.at[h],
                device_id=(right,), device_id_type=pl.DeviceIdType.MESH,
            )
            l = pltpu.make_async_remote_copy(
                src_ref=send_l, dst_ref=recv_l.at[h],
                send_sem=ss_l.at[h], recv_sem=rs_l.at[h],
                device_id=(left,), device_id_type=pl.DeviceIdType.MESH,
            )
            r.start()
            l.start()
            return r, l

        send_r[...] = part_a((my + N_DEV - 1) % N_DEV).astype(COMM_DTYPE)
        send_l[...] = part_b((my + 1) % N_DEV).astype(COMM_DTYPE)

        barrier_sem = pltpu.get_barrier_semaphore()
        for nbr in (left, right):
            pl.semaphore_signal(
                barrier_sem, inc=1,
                device_id=(nbr,), device_id_type=pl.DeviceIdType.MESH,
            )
        pl.semaphore_wait(barrier_sem, 2)

        cur_r, cur_l = start_hop(0)

        for h in range(N_DEV - 2):
            a = part_a((my + N_DEV - 2 - h) % N_DEV)
            b = part_b((my + 2 + h) % N_DEV)
            cur_r.wait_recv()
            cur_r.wait_send()
            send_r[...] = (a + recv_r[h].astype(jnp.float32)).astype(COMM_DTYPE)
            cur_l.wait_recv()
            cur_l.wait_send()
            send_l[...] = (b + recv_l[h].astype(jnp.float32)).astype(COMM_DTYPE)
            cur_r, cur_l = start_hop(h + 1)

        h = N_DEV - 2
        a = part_a(my)
        b = part_b(my)
        cur_r.wait_recv()
        cur_r.wait_send()
        cur_l.wait_recv()
        cur_l.wait_send()
        scale = sx_ref[0] * sw_ref[0]
        ya = (a + recv_r[h].astype(jnp.float32)) * scale
        yb = (b + recv_l[h].astype(jnp.float32)) * scale
        za = jnp.clip(ya, -60.0, 60.0)
        zb = jnp.clip(yb, -60.0, 60.0)
        out_ref[:, :n_half] = ya / (1.0 + jnp.exp(-za))
        out_ref[:, n_half:] = yb / (1.0 + jnp.exp(-zb))

    return pl.pallas_call(
        body,
        out_shape=jax.ShapeDtypeStruct((m_chunk, n_out), jnp.float32),
        in_specs=[
            pl.BlockSpec(memory_space=pltpu.VMEM),
            pl.BlockSpec(memory_space=pltpu.VMEM),
            pl.BlockSpec(memory_space=pltpu.SMEM),
            pl.BlockSpec(memory_space=pltpu.SMEM),
        ],
        out_specs=pl.BlockSpec(memory_space=pltpu.VMEM),
        scratch_shapes=[
            pltpu.VMEM((m_total, k_shard), jnp.float8_e4m3fn),
            pltpu.VMEM((k_shard, n_out), jnp.float8_e4m3fn),
            pltpu.VMEM((m_chunk, n_half), COMM_DTYPE),
            pltpu.VMEM((m_chunk, n_half), COMM_DTYPE),
            pltpu.VMEM((N_DEV - 1, m_chunk, n_half), COMM_DTYPE),
            pltpu.VMEM((N_DEV - 1, m_chunk, n_half), COMM_DTYPE),
            pltpu.SemaphoreType.DMA((N_DEV - 1,)),
            pltpu.SemaphoreType.DMA((N_DEV - 1,)),
            pltpu.SemaphoreType.DMA((N_DEV - 1,)),
            pltpu.SemaphoreType.DMA((N_DEV - 1,)),
        ],
        compiler_params=pltpu.CompilerParams(
            collective_id=0, vmem_limit_bytes=100 * 1024 * 1024
        ),
    )(x, w_mat, scale_x, scale_w)


# device time: 93549 ns/iter; 1.9476x vs baseline; 1.1462x over previous
import contextlib
import os

import jax
import jax.numpy as jnp
from jax import lax
from jax.experimental import pallas as pl
from jax.experimental.pallas import tpu as pltpu

N_DEV = 4
N_SUB = 2
COMM_DTYPE = jnp.bfloat16

_SCOPES = os.environ.get("KERNEL_SCOPES") == "1"


def _scope(name):
    return jax.named_scope(name) if _SCOPES else contextlib.nullcontext()


def kernel(x, w_mat, scale_x, scale_w):
    m_total, k_shard = x.shape
    _, n_out = w_mat.shape
    m_chunk = m_total // N_DEV
    n_half = n_out // 2
    m_sub = m_chunk // N_SUB

    def body(x_ref, w_ref, sx_ref, sw_ref, out_ref,
             x8_ref, w8_ref,
             send_r, send_l, recv_r, recv_l,
             ss_r, rs_r, ss_l, rs_l):
        my = lax.axis_index("i")
        left = (my + N_DEV - 1) % N_DEV
        right = (my + 1) % N_DEV

        barrier_sem = pltpu.get_barrier_semaphore()
        for nbr in (left, right):
            pl.semaphore_signal(
                barrier_sem, inc=1,
                device_id=(nbr,), device_id_type=pl.DeviceIdType.MESH,
            )

        with _scope("cast"):
            x8_ref[...] = x_ref[...].astype(jnp.float8_e4m3fn)
            w8_ref[...] = w_ref[...].astype(jnp.float8_e4m3fn)

        def part_a(c, s):
            rows = x8_ref[pl.ds(c * m_chunk + s * m_sub, m_sub), :]
            return jnp.dot(rows, w8_ref[:, :n_half],
                           preferred_element_type=jnp.float32)

        def part_b(c, s):
            rows = x8_ref[pl.ds(c * m_chunk + s * m_sub, m_sub), :]
            return jnp.dot(rows, w8_ref[:, n_half:],
                           preferred_element_type=jnp.float32)

        def start_sub(h, s):
            r = pltpu.make_async_remote_copy(
                src_ref=send_r.at[s], dst_ref=recv_r.at[h, s],
                send_sem=ss_r.at[h, s], recv_sem=rs_r.at[h, s],
                device_id=(right,), device_id_type=pl.DeviceIdType.MESH,
            )
            l = pltpu.make_async_remote_copy(
                src_ref=send_l.at[s], dst_ref=recv_l.at[h, s],
                send_sem=ss_l.at[h, s], recv_sem=rs_l.at[h, s],
                device_id=(left,), device_id_type=pl.DeviceIdType.MESH,
            )
            return r, l

        c_r0 = (my + N_DEV - 1) % N_DEV
        c_l0 = (my + 1) % N_DEV

        with _scope("dots0"):
            send_r[0] = part_a(c_r0, 0).astype(COMM_DTYPE)
            send_l[0] = part_b(c_l0, 0).astype(COMM_DTYPE)
        with _scope("barrier"):
            pl.semaphore_wait(barrier_sem, 2)
        rdmas = {}
        rdmas[(0, 0)] = start_sub(0, 0)
        rdmas[(0, 0)][0].start()
        rdmas[(0, 0)][1].start()
        with _scope("dots0b"):
            send_r[1] = part_a(c_r0, 1).astype(COMM_DTYPE)
            send_l[1] = part_b(c_l0, 1).astype(COMM_DTYPE)
        rdmas[(0, 1)] = start_sub(0, 1)
        rdmas[(0, 1)][0].start()
        rdmas[(0, 1)][1].start()

        for h in range(N_DEV - 2):
            c_r = (my + N_DEV - 2 - h) % N_DEV
            c_l = (my + 2 + h) % N_DEV
            with _scope(f"dots_h{h}"):
                a = [part_a(c_r, s) for s in range(N_SUB)]
                b = [part_b(c_l, s) for s in range(N_SUB)]
            for s in range(N_SUB):
                cur_r, cur_l = rdmas.pop((h, s))
                with _scope(f"waitR_h{h}s{s}"):
                    cur_r.wait_recv()
                    cur_r.wait_send()
                with _scope(f"storeR_h{h}s{s}"):
                    send_r[s] = (
                        a[s] + recv_r[h, s].astype(jnp.float32)
                    ).astype(COMM_DTYPE)
                nxt = start_sub(h + 1, s)
                nxt[0].start()
                with _scope(f"waitL_h{h}s{s}"):
                    cur_l.wait_recv()
                    cur_l.wait_send()
                with _scope(f"storeL_h{h}s{s}"):
                    send_l[s] = (
                        b[s] + recv_l[h, s].astype(jnp.float32)
                    ).astype(COMM_DTYPE)
                nxt[1].start()
                rdmas[(h + 1, s)] = nxt

        h = N_DEV - 2
        with _scope("dots_h2"):
            a = [part_a(my, s) for s in range(N_SUB)]
            b = [part_b(my, s) for s in range(N_SUB)]
        scale = sx_ref[0] * sw_ref[0]
        for s in range(N_SUB):
            cur_r, cur_l = rdmas.pop((h, s))
            with _scope(f"wait_h2s{s}"):
                cur_r.wait_recv()
                cur_r.wait_send()
                cur_l.wait_recv()
                cur_l.wait_send()
            with _scope(f"epilogue_s{s}"):
                ya = (a[s] + recv_r[h, s].astype(jnp.float32)) * scale
                yb = (b[s] + recv_l[h, s].astype(jnp.float32)) * scale
                za = jnp.clip(ya, -60.0, 60.0)
                zb = jnp.clip(yb, -60.0, 60.0)
                rows = pl.ds(s * m_sub, m_sub)
                out_ref[rows, :n_half] = ya / (1.0 + jnp.exp(-za))
                out_ref[rows, n_half:] = yb / (1.0 + jnp.exp(-zb))

    return pl.pallas_call(
        body,
        out_shape=jax.ShapeDtypeStruct((m_chunk, n_out), jnp.float32),
        in_specs=[
            pl.BlockSpec(memory_space=pltpu.VMEM),
            pl.BlockSpec(memory_space=pltpu.VMEM),
            pl.BlockSpec(memory_space=pltpu.SMEM),
            pl.BlockSpec(memory_space=pltpu.SMEM),
        ],
        out_specs=pl.BlockSpec(memory_space=pltpu.VMEM),
        scratch_shapes=[
            pltpu.VMEM((m_total, k_shard), jnp.float8_e4m3fn),
            pltpu.VMEM((k_shard, n_out), jnp.float8_e4m3fn),
            pltpu.VMEM((N_SUB, m_sub, n_half), COMM_DTYPE),
            pltpu.VMEM((N_SUB, m_sub, n_half), COMM_DTYPE),
            pltpu.VMEM((N_DEV - 1, N_SUB, m_sub, n_half), COMM_DTYPE),
            pltpu.VMEM((N_DEV - 1, N_SUB, m_sub, n_half), COMM_DTYPE),
            pltpu.SemaphoreType.DMA((N_DEV - 1, N_SUB)),
            pltpu.SemaphoreType.DMA((N_DEV - 1, N_SUB)),
            pltpu.SemaphoreType.DMA((N_DEV - 1, N_SUB)),
            pltpu.SemaphoreType.DMA((N_DEV - 1, N_SUB)),
        ],
        compiler_params=pltpu.CompilerParams(
            collective_id=0, vmem_limit_bytes=100 * 1024 * 1024
        ),
    )(x, w_mat, scale_x, scale_w)


# device time: 82493 ns/iter; 2.2086x vs baseline; 1.1340x over previous
import contextlib
import os

import jax
import jax.numpy as jnp
from jax import lax
from jax.experimental import pallas as pl
from jax.experimental.pallas import tpu as pltpu

N_DEV = 4
N_SUB = 2
COMM_DTYPE = jnp.bfloat16

_SCOPES = os.environ.get("KERNEL_SCOPES") == "1"


def _scope(name):
    return jax.named_scope(name) if _SCOPES else contextlib.nullcontext()


def kernel(x, w_mat, scale_x, scale_w):
    m_total, k_shard = x.shape
    _, n_out = w_mat.shape
    m_chunk = m_total // N_DEV
    n_half = n_out // 2
    m_sub = m_chunk // N_SUB

    def body(x_ref, w_ref, sx_ref, sw_ref, out_ref,
             x8_ref, w8_ref,
             send_r, send_l, recv_r, recv_l,
             send_r8, send_l8, recv_r8, recv_l8,
             ss_r, rs_r, ss_l, rs_l):
        my = lax.axis_index("i")
        left = (my + N_DEV - 1) % N_DEV
        right = (my + 1) % N_DEV

        barrier_sem = pltpu.get_barrier_semaphore()
        for nbr in (left, right):
            pl.semaphore_signal(
                barrier_sem, inc=1,
                device_id=(nbr,), device_id_type=pl.DeviceIdType.MESH,
            )

        with _scope("cast"):
            x8_ref[...] = x_ref[...].astype(jnp.float8_e4m3fn)
            w8_ref[...] = w_ref[...].astype(jnp.float8_e4m3fn)

        def part_a(c, s):
            rows = x8_ref[pl.ds(c * m_chunk + s * m_sub, m_sub), :]
            return jnp.dot(rows, w8_ref[:, :n_half],
                           preferred_element_type=jnp.float32)

        def part_b(c, s):
            rows = x8_ref[pl.ds(c * m_chunk + s * m_sub, m_sub), :]
            return jnp.dot(rows, w8_ref[:, n_half:],
                           preferred_element_type=jnp.float32)

        def start_sub(h, s):
            sr, dr = (send_r8, recv_r8) if h == 0 else (send_r, recv_r)
            sl, dl = (send_l8, recv_l8) if h == 0 else (send_l, recv_l)
            dst_r = dr.at[s] if h == 0 else dr.at[h, s]
            dst_l = dl.at[s] if h == 0 else dl.at[h, s]
            r = pltpu.make_async_remote_copy(
                src_ref=sr.at[s], dst_ref=dst_r,
                send_sem=ss_r.at[h, s], recv_sem=rs_r.at[h, s],
                device_id=(right,), device_id_type=pl.DeviceIdType.MESH,
            )
            l = pltpu.make_async_remote_copy(
                src_ref=sl.at[s], dst_ref=dst_l,
                send_sem=ss_l.at[h, s], recv_sem=rs_l.at[h, s],
                device_id=(left,), device_id_type=pl.DeviceIdType.MESH,
            )
            return r, l

        c_r0 = (my + N_DEV - 1) % N_DEV
        c_l0 = (my + 1) % N_DEV

        with _scope("dots0"):
            send_r8[0] = part_a(c_r0, 0).astype(jnp.float8_e4m3fn)
            send_l8[0] = part_b(c_l0, 0).astype(jnp.float8_e4m3fn)
        with _scope("barrier"):
            pl.semaphore_wait(barrier_sem, 2)
        rdmas = {}
        rdmas[(0, 0)] = start_sub(0, 0)
        rdmas[(0, 0)][0].start()
        rdmas[(0, 0)][1].start()
        with _scope("dots0b"):
            send_r8[1] = part_a(c_r0, 1).astype(jnp.float8_e4m3fn)
            send_l8[1] = part_b(c_l0, 1).astype(jnp.float8_e4m3fn)
        rdmas[(0, 1)] = start_sub(0, 1)
        rdmas[(0, 1)][0].start()
        rdmas[(0, 1)][1].start()

        for h in range(N_DEV - 2):
            c_r = (my + N_DEV - 2 - h) % N_DEV
            c_l = (my + 2 + h) % N_DEV
            with _scope(f"dots_h{h}"):
                a = [part_a(c_r, s) for s in range(N_SUB)]
                b = [part_b(c_l, s) for s in range(N_SUB)]
            for s in range(N_SUB):
                cur_r, cur_l = rdmas.pop((h, s))
                with _scope(f"waitR_h{h}s{s}"):
                    cur_r.wait_recv()
                    cur_r.wait_send()
                with _scope(f"storeR_h{h}s{s}"):
                    rr = recv_r8[s] if h == 0 else recv_r[h, s]
                    send_r[s] = (
                        a[s] + rr.astype(jnp.float32)
                    ).astype(COMM_DTYPE)
                nxt = start_sub(h + 1, s)
                nxt[0].start()
                with _scope(f"waitL_h{h}s{s}"):
                    cur_l.wait_recv()
                    cur_l.wait_send()
                with _scope(f"storeL_h{h}s{s}"):
                    rl = recv_l8[s] if h == 0 else recv_l[h, s]
                    send_l[s] = (
                        b[s] + rl.astype(jnp.float32)
                    ).astype(COMM_DTYPE)
                nxt[1].start()
                rdmas[(h + 1, s)] = nxt

        h = N_DEV - 2
        with _scope("dots_h2"):
            a = [part_a(my, s) for s in range(N_SUB)]
            b = [part_b(my, s) for s in range(N_SUB)]
        scale = sx_ref[0] * sw_ref[0]
        for s in range(N_SUB):
            cur_r, cur_l = rdmas.pop((h, s))
            with _scope(f"wait_h2s{s}"):
                cur_r.wait_recv()
                cur_r.wait_send()
                cur_l.wait_recv()
                cur_l.wait_send()
            with _scope(f"epilogue_s{s}"):
                ya = (a[s] + recv_r[h, s].astype(jnp.float32)) * scale
                yb = (b[s] + recv_l[h, s].astype(jnp.float32)) * scale
                za = jnp.clip(ya, -60.0, 60.0)
                zb = jnp.clip(yb, -60.0, 60.0)
                rows = pl.ds(s * m_sub, m_sub)
                out_ref[rows, :n_half] = ya / (1.0 + jnp.exp(-za))
                out_ref[rows, n_half:] = yb / (1.0 + jnp.exp(-zb))

    return pl.pallas_call(
        body,
        out_shape=jax.ShapeDtypeStruct((m_chunk, n_out), jnp.float32),
        in_specs=[
            pl.BlockSpec(memory_space=pltpu.VMEM),
            pl.BlockSpec(memory_space=pltpu.VMEM),
            pl.BlockSpec(memory_space=pltpu.SMEM),
            pl.BlockSpec(memory_space=pltpu.SMEM),
        ],
        out_specs=pl.BlockSpec(memory_space=pltpu.VMEM),
        scratch_shapes=[
            pltpu.VMEM((m_total, k_shard), jnp.float8_e4m3fn),
            pltpu.VMEM((k_shard, n_out), jnp.float8_e4m3fn),
            pltpu.VMEM((N_SUB, m_sub, n_half), COMM_DTYPE),
            pltpu.VMEM((N_SUB, m_sub, n_half), COMM_DTYPE),
            pltpu.VMEM((N_DEV - 1, N_SUB, m_sub, n_half), COMM_DTYPE),
            pltpu.VMEM((N_DEV - 1, N_SUB, m_sub, n_half), COMM_DTYPE),
            pltpu.VMEM((N_SUB, m_sub, n_half), jnp.float8_e4m3fn),
            pltpu.VMEM((N_SUB, m_sub, n_half), jnp.float8_e4m3fn),
            pltpu.VMEM((N_SUB, m_sub, n_half), jnp.float8_e4m3fn),
            pltpu.VMEM((N_SUB, m_sub, n_half), jnp.float8_e4m3fn),
            pltpu.SemaphoreType.DMA((N_DEV - 1, N_SUB)),
            pltpu.SemaphoreType.DMA((N_DEV - 1, N_SUB)),
            pltpu.SemaphoreType.DMA((N_DEV - 1, N_SUB)),
            pltpu.SemaphoreType.DMA((N_DEV - 1, N_SUB)),
        ],
        compiler_params=pltpu.CompilerParams(
            collective_id=0, vmem_limit_bytes=100 * 1024 * 1024
        ),
    )(x, w_mat, scale_x, scale_w)
